# baseline (speedup 1.0000x reference)
import functools

import jax
import jax.numpy as jnp
from jax import lax
from jax.experimental import pallas as pl
from jax.experimental.pallas import tpu as pltpu
from jax.experimental.pallas import tpu_sc as plsc

_BLOCKS = 12
_BS = 8
_N = 677
_NUM_POINTS = 100
_NUM_GT = 4
_SCALE = 4
_NUM_PATCHES = _N - 1 - _NUM_POINTS
_KK = int(_N * _N * 0.5)

_IMG = _N * _N
_TOTAL = _BLOCKS * _BS * _IMG
_CH = 32768
_NCH = 14
_WIN = _CH * _NCH
_NBUCKET = 32768
_NWORKER = 32


def _sc_threshold_kernel(flat_hbm, thr_hbm, buf0, buf1, hist, tvec,
                         sem0, sem1):
    cid = lax.axis_index("c")
    sid = lax.axis_index("s")
    wid = cid * 16 + sid
    base = (64 + wid) * _IMG
    wb = jnp.minimum((base // 8) * 8, _TOTAL - _WIN)
    lo = base - wb
    hi = lo + _IMG

    zeros = jnp.zeros((16,), jnp.float32)

    @plsc.parallel_loop(0, _NBUCKET, 16, unroll=4)
    def _zero(i):
        hist[pl.ds(i, 16)] = zeros

    ones = jnp.ones((16,), jnp.float32)
    iota = lax.broadcasted_iota(jnp.int32, (16,), 0)

    def consume(buf, coff, masked):
        if masked:
            @plsc.parallel_loop(0, _CH, 16, unroll=4)
            def _body(i):
                x = buf[pl.ds(i, 16)]
                bidx = (x * float(_NBUCKET)).astype(jnp.int32)
                ridx = coff + i + iota
                ok = (ridx >= lo) & (ridx < hi)
                plsc.addupdate_scatter(hist, [bidx], ones, mask=ok)
        else:
            @plsc.parallel_loop(0, _CH, 16, unroll=4)
            def _body(i):
                x = buf[pl.ds(i, 16)]
                bidx = (x * float(_NBUCKET)).astype(jnp.int32)
                plsc.addupdate_scatter(hist, [bidx], ones)

    bufs = (buf0, buf1)
    sems = (sem0, sem1)
    dma = pltpu.async_copy(flat_hbm.at[pl.ds(wb, _CH)], buf0, sem0)
    for c in range(_NCH):
        dma.wait()
        if c + 1 < _NCH:
            dma = pltpu.async_copy(
                flat_hbm.at[pl.ds(wb + (c + 1) * _CH, _CH)],
                bufs[(c + 1) % 2], sems[(c + 1) % 2])
        consume(bufs[c % 2], c * _CH, c == 0 or c == _NCH - 1)

    def sbody(i, carry):
        cum, bsel = carry
        h = hist[pl.ds(i * 16, 16)]
        cs = plsc.cumsum(h)
        cnt = jnp.sum((cum + cs <= float(_KK)).astype(jnp.float32))
        bsel = jnp.where(cnt > 0.0, i * 16 + cnt.astype(jnp.int32), bsel)
        return cum + jnp.sum(h), bsel

    _, bsel = lax.fori_loop(0, _NBUCKET // 16, sbody,
                            (jnp.float32(0.0), jnp.int32(0)))

    t = bsel.astype(jnp.float32) * (1.0 / float(_NBUCKET))
    tvec[...] = jnp.full((16,), 0.0, jnp.float32) + t
    pltpu.sync_copy(tvec, thr_hbm.at[wid])


def _sc_thresholds(attns_flat):
    mesh = plsc.VectorSubcoreMesh(core_axis_name="c", subcore_axis_name="s")
    k = functools.partial(
        pl.kernel,
        mesh=mesh,
        compiler_params=pltpu.CompilerParams(needs_layout_passes=False),
        out_type=jax.ShapeDtypeStruct((_NWORKER, 16), jnp.float32),
        scratch_types=[
            pltpu.VMEM((_CH,), jnp.float32),
            pltpu.VMEM((_CH,), jnp.float32),
            pltpu.VMEM((_NBUCKET,), jnp.float32),
            pltpu.VMEM((16,), jnp.float32),
            pltpu.SemaphoreType.DMA,
            pltpu.SemaphoreType.DMA,
        ],
    )(_sc_threshold_kernel)
    return k(attns_flat)


def _rollout_kernel(pos_ref, thr_ref, attn_ref, out_ref, w_ref):
    b = pl.program_id(0)
    j = pl.program_id(1)

    a = attn_ref[0, 0]
    tb = thr_ref[(_SCALE - 1 - j) * _BS + b]
    a_bits = jax.lax.bitcast_convert_type(a, jnp.int32)
    f = jnp.where(a_bits >= tb, a, 0.0)

    inv = 1.0 / (jnp.sum(f, axis=1) + 1.0)

    @pl.when(j == 0)
    def _start():
        for g in range(_NUM_GT):
            r = _N - _NUM_POINTS + pos_ref[b, g]
            raw = attn_ref[0, 0, pl.ds(r, 1), :]
            rbits = jax.lax.bitcast_convert_type(raw, jnp.int32)
            row = jnp.where(rbits >= tb, raw, 0.0)
            col = jax.lax.broadcasted_iota(jnp.int32, (1, _N), 1)
            row = row + jnp.where(col == r, 1.0, 0.0)
            row = row * (1.0 / jnp.sum(row))
            w_ref[g, :] = row[0, :]
            out_ref[0, 0, g, :] = row[0, 1 : 1 + _NUM_PATCHES]

    @pl.when(j > 0)
    def _step():
        w = w_ref[0:_NUM_GT, :]
        u = w * inv[None, :]
        w_new = jnp.dot(u, f, preferred_element_type=jnp.float32) + u
        w_ref[0:_NUM_GT, :] = w_new
        out_ref[0, 0, :, :] = w_new[:, 1 : 1 + _NUM_PATCHES]


def kernel(attns_maps, pos_inds):
    pos = pos_inds.astype(jnp.int32)

    thr = _sc_thresholds(attns_maps.reshape(-1))
    thr_bits = jax.lax.bitcast_convert_type(thr[:, 0], jnp.int32)

    grid_spec = pltpu.PrefetchScalarGridSpec(
        num_scalar_prefetch=2,
        grid=(_BS, _SCALE),
        in_specs=[
            pl.BlockSpec(
                (1, 1, _N, _N),
                lambda b, j, pos_ref, thr_ref: (_BLOCKS - 1 - j, b, 0, 0),
            ),
        ],
        out_specs=pl.BlockSpec(
            (1, 1, _NUM_GT, _NUM_PATCHES),
            lambda b, j, pos_ref, thr_ref: (b, _SCALE - 1 - j, 0, 0),
        ),
        scratch_shapes=[pltpu.VMEM((8, _N), jnp.float32)],
    )

    out = pl.pallas_call(
        _rollout_kernel,
        grid_spec=grid_spec,
        out_shape=jax.ShapeDtypeStruct(
            (_BS, _SCALE, _NUM_GT, _NUM_PATCHES), jnp.float32
        ),
    )(pos, thr_bits, attns_maps)

    return jnp.transpose(out, (0, 2, 1, 3)).reshape(
        _BS * _NUM_GT, _SCALE, _NUM_PATCHES
    )


# speedup vs baseline: 7.5973x; 7.5973x over previous
import functools

import jax
import jax.numpy as jnp
from jax import lax
from jax.experimental import pallas as pl
from jax.experimental.pallas import tpu as pltpu
from jax.experimental.pallas import tpu_sc as plsc

_BLOCKS = 12
_BS = 8
_N = 677
_NUM_POINTS = 100
_NUM_GT = 4
_SCALE = 4
_NUM_PATCHES = _N - 1 - _NUM_POINTS
_KK = int(_N * _N * 0.5)

_NBUCKET = 32768
_NWORKER = 32
_RCH = 48
_NFULL = _N // _RCH
_RTAIL = _N - _NFULL * _RCH
_CFULL = (_N // 16) * 16


def _sc_threshold_kernel(attns_hbm, thr_hbm, buf0, buf1, tbuf, hist, tvec,
                         sem0, sem1):
    cid = lax.axis_index("c")
    sid = lax.axis_index("s")
    wid = cid * 16 + sid
    blk = 8 + wid // 8
    b = wid % 8

    zeros = jnp.zeros((16,), jnp.float32)

    @plsc.parallel_loop(0, _NBUCKET, 16, unroll=4)
    def _zero(i):
        hist[pl.ds(i, 16)] = zeros

    ones = jnp.ones((16,), jnp.float32)
    iota = lax.broadcasted_iota(jnp.int32, (16,), 0)
    tailmask = iota >= (16 - (_N - _CFULL))

    def consume(buf2, nrows):
        def rbody(r, cc):
            @plsc.parallel_loop(0, _CFULL, 16, unroll=4)
            def _b(i):
                x = buf2[r, pl.ds(i, 16)]
                bidx = (x * float(_NBUCKET)).astype(jnp.int32)
                plsc.addupdate_scatter(hist, [bidx], ones)
            x = buf2[r, pl.ds(_N - 16, 16)]
            bidx = (x * float(_NBUCKET)).astype(jnp.int32)
            plsc.addupdate_scatter(hist, [bidx], ones, mask=tailmask)
            return cc
        lax.fori_loop(0, nrows, rbody, 0)

    bufs = (buf0, buf1)
    sems = (sem0, sem1)
    dma = pltpu.async_copy(attns_hbm.at[blk, b, pl.ds(0, _RCH), :], buf0, sem0)
    for c in range(_NFULL):
        dma.wait()
        if c + 1 < _NFULL:
            dma = pltpu.async_copy(
                attns_hbm.at[blk, b, pl.ds((c + 1) * _RCH, _RCH), :],
                bufs[(c + 1) % 2], sems[(c + 1) % 2])
        elif _RTAIL:
            dma = pltpu.async_copy(
                attns_hbm.at[blk, b, pl.ds(_NFULL * _RCH, _RTAIL), :],
                tbuf, sems[(c + 1) % 2])
        consume(bufs[c % 2], _RCH)
    if _RTAIL:
        dma.wait()
        consume(tbuf, _RTAIL)

    def sbody(i, carry):
        cum, bsel = carry
        h = hist[pl.ds(i * 16, 16)]
        cs = plsc.cumsum(h)
        cnt = jnp.sum((cum + cs <= float(_KK)).astype(jnp.float32))
        bsel = jnp.where(cnt > 0.0, i * 16 + cnt.astype(jnp.int32), bsel)
        return cum + jnp.sum(h), bsel

    _, bsel = lax.fori_loop(0, _NBUCKET // 16, sbody,
                            (jnp.float32(0.0), jnp.int32(0)))

    t = bsel.astype(jnp.float32) * (1.0 / float(_NBUCKET))
    tvec[...] = jnp.full((16,), 0.0, jnp.float32) + t
    pltpu.sync_copy(tvec, thr_hbm.at[wid])


def _sc_thresholds(attns_maps):
    mesh = plsc.VectorSubcoreMesh(core_axis_name="c", subcore_axis_name="s")
    k = functools.partial(
        pl.kernel,
        mesh=mesh,
        compiler_params=pltpu.CompilerParams(needs_layout_passes=False),
        out_type=jax.ShapeDtypeStruct((_NWORKER, 16), jnp.float32),
        scratch_types=[
            pltpu.VMEM((_RCH, _N), jnp.float32),
            pltpu.VMEM((_RCH, _N), jnp.float32),
            pltpu.VMEM((_RTAIL, _N), jnp.float32),
            pltpu.VMEM((_NBUCKET,), jnp.float32),
            pltpu.VMEM((16,), jnp.float32),
            pltpu.SemaphoreType.DMA,
            pltpu.SemaphoreType.DMA,
        ],
    )(_sc_threshold_kernel)
    return k(attns_maps)


def _rollout_kernel(pos_ref, thr_ref, attn_ref, out_ref, w_ref):
    b = pl.program_id(0)
    j = pl.program_id(1)

    a = attn_ref[0, 0]
    tb = thr_ref[(_SCALE - 1 - j) * _BS + b]
    a_bits = jax.lax.bitcast_convert_type(a, jnp.int32)
    f = jnp.where(a_bits >= tb, a, 0.0)

    inv = 1.0 / (jnp.sum(f, axis=1) + 1.0)

    @pl.when(j == 0)
    def _start():
        for g in range(_NUM_GT):
            r = _N - _NUM_POINTS + pos_ref[b, g]
            raw = attn_ref[0, 0, pl.ds(r, 1), :]
            rbits = jax.lax.bitcast_convert_type(raw, jnp.int32)
            row = jnp.where(rbits >= tb, raw, 0.0)
            col = jax.lax.broadcasted_iota(jnp.int32, (1, _N), 1)
            row = row + jnp.where(col == r, 1.0, 0.0)
            row = row * (1.0 / jnp.sum(row))
            w_ref[g, :] = row[0, :]
            out_ref[0, 0, g, :] = row[0, 1 : 1 + _NUM_PATCHES]

    @pl.when(j > 0)
    def _step():
        w = w_ref[0:_NUM_GT, :]
        u = w * inv[None, :]
        w_new = jnp.dot(u, f, preferred_element_type=jnp.float32) + u
        w_ref[0:_NUM_GT, :] = w_new
        out_ref[0, 0, :, :] = w_new[:, 1 : 1 + _NUM_PATCHES]


def kernel(attns_maps, pos_inds):
    pos = pos_inds.astype(jnp.int32)

    thr = _sc_thresholds(attns_maps)
    thr_bits = jax.lax.bitcast_convert_type(thr[:, 0], jnp.int32)

    grid_spec = pltpu.PrefetchScalarGridSpec(
        num_scalar_prefetch=2,
        grid=(_BS, _SCALE),
        in_specs=[
            pl.BlockSpec(
                (1, 1, _N, _N),
                lambda b, j, pos_ref, thr_ref: (_BLOCKS - 1 - j, b, 0, 0),
            ),
        ],
        out_specs=pl.BlockSpec(
            (1, 1, _NUM_GT, _NUM_PATCHES),
            lambda b, j, pos_ref, thr_ref: (b, _SCALE - 1 - j, 0, 0),
        ),
        scratch_shapes=[pltpu.VMEM((8, _N), jnp.float32)],
    )

    out = pl.pallas_call(
        _rollout_kernel,
        grid_spec=grid_spec,
        out_shape=jax.ShapeDtypeStruct(
            (_BS, _SCALE, _NUM_GT, _NUM_PATCHES), jnp.float32
        ),
    )(pos, thr_bits, attns_maps)

    return jnp.transpose(out, (0, 2, 1, 3)).reshape(
        _BS * _NUM_GT, _SCALE, _NUM_PATCHES
    )


# speedup vs baseline: 12.4009x; 1.6323x over previous
import functools

import jax
import jax.numpy as jnp
from jax import lax
from jax.experimental import pallas as pl
from jax.experimental.pallas import tpu as pltpu
from jax.experimental.pallas import tpu_sc as plsc

_BLOCKS = 12
_BS = 8
_N = 677
_NUM_POINTS = 100
_NUM_GT = 4
_SCALE = 4
_NUM_PATCHES = _N - 1 - _NUM_POINTS
_KK = int(_N * _N * 0.5)

_NBUCKET = 32768
_NWORKER = 32
_RCH = 48
_NFULL = _N // _RCH
_RTAIL = _N - _NFULL * _RCH
_CFULL = (_N // 16) * 16


def _sc_threshold_kernel(attns_hbm, thr_hbm, buf0, buf1, tbuf, hist, tvec,
                         sem0, sem1):
    cid = lax.axis_index("c")
    sid = lax.axis_index("s")
    wid = cid * 16 + sid
    blk = _BLOCKS - _SCALE + wid // 8
    b = wid % 8

    zeros = jnp.zeros((16,), jnp.float32)
    ones = jnp.ones((16,), jnp.float32)

    @plsc.parallel_loop(0, _NBUCKET, 16, unroll=4)
    def _zero(i):
        hist[pl.ds(i, 16)] = zeros

    iota = lax.broadcasted_iota(jnp.int32, (16,), 0)
    tailmask = iota >= (16 - (_N - _CFULL))

    def consume(buf2, nrows):
        def rbody(r, cc):
            @plsc.parallel_loop(0, _CFULL, 16, unroll=4)
            def _b(i):
                x = buf2[r, pl.ds(i, 16)]
                bidx = (x * float(_NBUCKET)).astype(jnp.int32)
                plsc.addupdate_scatter(hist, [bidx], ones)
            x = buf2[r, pl.ds(_N - 16, 16)]
            bidx = (x * float(_NBUCKET)).astype(jnp.int32)
            plsc.addupdate_scatter(hist, [bidx], ones, mask=tailmask)
            return cc
        lax.fori_loop(0, nrows, rbody, 0)

    bufs = (buf0, buf1)
    sems = (sem0, sem1)
    dma = pltpu.async_copy(attns_hbm.at[blk, pl.ds(0, _RCH), b, :], buf0, sem0)
    for c in range(_NFULL):
        dma.wait()
        if c + 1 < _NFULL:
            dma = pltpu.async_copy(
                attns_hbm.at[blk, pl.ds((c + 1) * _RCH, _RCH), b, :],
                bufs[(c + 1) % 2], sems[(c + 1) % 2])
        elif _RTAIL:
            dma = pltpu.async_copy(
                attns_hbm.at[blk, pl.ds(_NFULL * _RCH, _RTAIL), b, :],
                tbuf, sems[(c + 1) % 2])
        consume(bufs[c % 2], _RCH)
    if _RTAIL:
        dma.wait()
        consume(tbuf, _RTAIL)

    def sbody(i, carry):
        cum, bsel = carry
        h = hist[pl.ds(i * 16, 16)]
        cs = plsc.cumsum(h)
        cnt = jnp.sum((cum + cs <= float(_KK)).astype(jnp.float32))
        bsel = jnp.where(cnt > 0.0, i * 16 + cnt.astype(jnp.int32), bsel)
        return cum + jnp.sum(h), bsel

    _, bsel = lax.fori_loop(0, _NBUCKET // 16, sbody,
                            (jnp.float32(0.0), jnp.int32(0)))

    t = bsel.astype(jnp.float32) * (1.0 / float(_NBUCKET))
    tvec[...] = jnp.full((16,), 0.0, jnp.float32) + t
    pltpu.sync_copy(tvec, thr_hbm.at[wid])


def _sc_thresholds(attns_maps):
    mesh = plsc.VectorSubcoreMesh(core_axis_name="c", subcore_axis_name="s")
    k = functools.partial(
        pl.kernel,
        mesh=mesh,
        compiler_params=pltpu.CompilerParams(needs_layout_passes=False),
        out_type=jax.ShapeDtypeStruct((_NWORKER, 16), jnp.float32),
        scratch_types=[
            pltpu.VMEM((_RCH, _N), jnp.float32),
            pltpu.VMEM((_RCH, _N), jnp.float32),
            pltpu.VMEM((_RTAIL, _N), jnp.float32),
            pltpu.VMEM((_NBUCKET,), jnp.float32),
            pltpu.VMEM((16,), jnp.float32),
            pltpu.SemaphoreType.DMA,
            pltpu.SemaphoreType.DMA,
        ],
    )(_sc_threshold_kernel)
    return k(attns_maps)


def _rollout_kernel(pos_ref, thr_ref, attn_ref, out_ref, w_ref):
    b = pl.program_id(0)
    j = pl.program_id(1)

    a = attn_ref[0, 0]
    tb = thr_ref[(_SCALE - 1 - j) * _BS + b]
    a_bits = jax.lax.bitcast_convert_type(a, jnp.int32)
    f = jnp.where(a_bits >= tb, a, 0.0)

    inv = 1.0 / (jnp.sum(f, axis=1) + 1.0)

    @pl.when(j == 0)
    def _start():
        for g in range(_NUM_GT):
            r = _N - _NUM_POINTS + pos_ref[b, g]
            raw = attn_ref[0, 0, pl.ds(r, 1), :]
            rbits = jax.lax.bitcast_convert_type(raw, jnp.int32)
            row = jnp.where(rbits >= tb, raw, 0.0)
            col = jax.lax.broadcasted_iota(jnp.int32, (1, _N), 1)
            row = row + jnp.where(col == r, 1.0, 0.0)
            row = row * (1.0 / jnp.sum(row))
            w_ref[g, :] = row[0, :]
            out_ref[0, 0, g, :] = row[0, 1 : 1 + _NUM_PATCHES]

    @pl.when(j > 0)
    def _step():
        w = w_ref[0:_NUM_GT, :]
        u = w * inv[None, :]
        w_new = jnp.dot(u, f, preferred_element_type=jnp.float32) + u
        w_ref[0:_NUM_GT, :] = w_new
        out_ref[0, 0, :, :] = w_new[:, 1 : 1 + _NUM_PATCHES]


def kernel(attns_maps, pos_inds):
    pos = pos_inds.astype(jnp.int32)
    xs = attns_maps[_BLOCKS - _SCALE :]

    xt = jnp.transpose(attns_maps, (0, 2, 1, 3))
    thr = _sc_thresholds(xt)
    thr_bits = jax.lax.bitcast_convert_type(thr[:, 0], jnp.int32)

    grid_spec = pltpu.PrefetchScalarGridSpec(
        num_scalar_prefetch=2,
        grid=(_BS, _SCALE),
        in_specs=[
            pl.BlockSpec(
                (1, 1, _N, _N),
                lambda b, j, pos_ref, thr_ref: (_SCALE - 1 - j, b, 0, 0),
            ),
        ],
        out_specs=pl.BlockSpec(
            (1, 1, _NUM_GT, _NUM_PATCHES),
            lambda b, j, pos_ref, thr_ref: (b, _SCALE - 1 - j, 0, 0),
        ),
        scratch_shapes=[pltpu.VMEM((8, _N), jnp.float32)],
    )

    out = pl.pallas_call(
        _rollout_kernel,
        grid_spec=grid_spec,
        out_shape=jax.ShapeDtypeStruct(
            (_BS, _SCALE, _NUM_GT, _NUM_PATCHES), jnp.float32
        ),
    )(pos, thr_bits, xs)

    return jnp.transpose(out, (0, 2, 1, 3)).reshape(
        _BS * _NUM_GT, _SCALE, _NUM_PATCHES
    )
